# baseline (speedup 1.0000x reference)
import jax
import jax.numpy as jnp
import numpy as np
from jax.experimental import pallas as pl
from jax.experimental.pallas import tpu as pltpu

N = 500000
GRAD_THRESHOLD = 0.5
MIN_OPACITY = 0.05
LOG2 = float(np.log(2.0))

LANES = 128
MB = 4096
MCH = (N + MB - 1) // MB
MSUB = MB // LANES
BBLK = 2000
NBLK = N // BBLK


def _median_kernel(sc_ref, gc_ref, thr_ref, gthr_ref, sn2_ref):
    g = pl.program_id(0)
    e = jnp.exp(sc_ref[...])
    sn2 = jnp.sum(e * e, axis=1, keepdims=True)
    tile = sn2.reshape(MSUB, LANES)
    lin = (g * MB + LANES * jax.lax.broadcasted_iota(jnp.int32,
                                                     (MSUB, LANES), 0)
           + jax.lax.broadcasted_iota(jnp.int32, (MSUB, LANES), 1))
    tile = jnp.where(lin < N, tile, jnp.inf)
    sn2_ref[pl.ds(pl.multiple_of(g * MSUB, MSUB), MSUB), :] = tile

    gc = jnp.maximum(gc_ref[...].astype(jnp.float32), 1.0)
    gthr_ref[...] = (GRAD_THRESHOLD * gc).reshape(MB, 1)

    @pl.when(g == MCH - 1)
    def _select():
        bits = jax.lax.bitcast_convert_type(sn2_ref[...], jnp.int32)
        k0 = N // 2 - 1
        k1 = N // 2

        def body(i, carry):
            p0, p1 = carry
            b = 30 - i
            c0 = p0 | (1 << b)
            c1 = p1 | (1 << b)
            n0 = jnp.sum((bits < c0).astype(jnp.int32))
            n1 = jnp.sum((bits < c1).astype(jnp.int32))
            return (jnp.where(n0 <= k0, c0, p0), jnp.where(n1 <= k1, c1, p1))

        t0, t1 = jax.lax.fori_loop(0, 31, body, (0, 0))
        v0 = jax.lax.bitcast_convert_type(t0, jnp.float32)
        v1 = jax.lax.bitcast_convert_type(t1, jnp.float32)
        thr_ref[...] = jnp.full((1, 1), 0.5 * (jnp.sqrt(v0) + jnp.sqrt(v1)),
                                jnp.float32)


def _main_kernel(thr_ref, gthr_ref, pos_ref, sc_ref, rot_ref, op_ref,
                 dc_ref, rest_ref, ga_ref, noise_ref, out_ref,
                 p_scr, asc_scr, keep_scr, clone_scr, split_scr):
    s = pl.program_id(1)

    @pl.when(s == 0)
    def _compute():
        thr = thr_ref[0, 0]
        ga = ga_ref[...]
        gn = jnp.sqrt(jnp.sum(ga * ga, axis=1, keepdims=True))
        large = gn >= gthr_ref[...]
        sc = sc_ref[...]
        asc = jnp.exp(sc)
        asc_scr[...] = asc
        sn = jnp.sqrt(jnp.sum(asc * asc, axis=1, keepdims=True))
        clone = large & (sn <= thr)
        split = large & (sn > thr)
        act_op = jax.nn.sigmoid(op_ref[...])
        keep = jnp.logical_not((act_op < MIN_OPACITY) | split)
        keep_scr[...] = keep.astype(jnp.float32)
        clone_scr[...] = clone.astype(jnp.float32)
        split_scr[...] = split.astype(jnp.float32)
        P = jnp.concatenate([pos_ref[...], sc, rot_ref[...], op_ref[...],
                             dc_ref[...], rest_ref[...]], axis=1)
        p_scr[...] = P
        out_ref[...] = jnp.where(keep, P, 0.0)

    @pl.when(s == 1)
    def _clone():
        out_ref[...] = jnp.where(clone_scr[...] > 0.0, p_scr[...], 0.0)

    @pl.when(s >= 2)
    def _split():
        sp_pos = pos_ref[...] + noise_ref[s - 2] * asc_scr[...]
        front = jnp.concatenate([sp_pos, sc_ref[...] - LOG2], axis=1)
        Pi = jnp.concatenate([front, p_scr[:, 6:23]], axis=1)
        out_ref[...] = jnp.where(split_scr[...] > 0.0, Pi, 0.0)


def kernel(positions, scales, rotations, opacities, sh_dc, sh_rest,
           grad_accum, grad_count, split_noise):
    f32 = jnp.float32
    thr, gthr = pl.pallas_call(
        _median_kernel,
        grid=(MCH,),
        in_specs=[
            pl.BlockSpec((MB, 3), lambda g: (g, 0)),
            pl.BlockSpec((MB,), lambda g: (g,)),
        ],
        out_specs=(
            pl.BlockSpec((1, 1), lambda g: (0, 0)),
            pl.BlockSpec((MB, 1), lambda g: (g, 0)),
        ),
        out_shape=(
            jax.ShapeDtypeStruct((1, 1), f32),
            jax.ShapeDtypeStruct((N, 1), f32),
        ),
        scratch_shapes=[pltpu.VMEM((MCH * MSUB, LANES), f32)],
    )(scales, grad_count)

    B = BBLK

    def row_spec(w):
        return pl.BlockSpec((B, w), lambda i, s: (i, 0))

    out = pl.pallas_call(
        _main_kernel,
        grid=(NBLK, 4),
        in_specs=[
            pl.BlockSpec((1, 1), lambda i, s: (0, 0)),
            row_spec(1),
            row_spec(3),
            row_spec(3),
            row_spec(4),
            row_spec(1),
            row_spec(3),
            row_spec(9),
            row_spec(2),
            pl.BlockSpec((2, B, 3), lambda i, s: (0, i, 0)),
        ],
        out_specs=pl.BlockSpec((B, 23), lambda i, s: (s * NBLK + i, 0)),
        out_shape=jax.ShapeDtypeStruct((4 * N, 23), f32),
        scratch_shapes=[
            pltpu.VMEM((B, 23), f32),
            pltpu.VMEM((B, 3), f32),
            pltpu.VMEM((B, 1), f32),
            pltpu.VMEM((B, 1), f32),
            pltpu.VMEM((B, 1), f32),
        ],
    )(thr, gthr, positions, scales, rotations, opacities, sh_dc, sh_rest,
      grad_accum, split_noise)
    return out


# speedup vs baseline: 1.5028x; 1.5028x over previous
import jax
import jax.numpy as jnp
import numpy as np
from jax.experimental import pallas as pl
from jax.experimental.pallas import tpu as pltpu

N = 500000
GRAD_THRESHOLD = 0.5
MIN_OPACITY = 0.05
LOG2 = float(np.log(2.0))

LANES = 128
MB = 4096
MCH = (N + MB - 1) // MB
MSUB = MB // LANES
BBLK = 2048
NBLK = (N + BBLK - 1) // BBLK


def _median_kernel(sc_ref, thr_ref, sn2_ref):
    g = pl.program_id(0)
    e = jnp.exp(sc_ref[...])
    sn2 = jnp.sum(e * e, axis=1, keepdims=True)
    tile = sn2.reshape(MSUB, LANES)
    lin = (g * MB + LANES * jax.lax.broadcasted_iota(jnp.int32,
                                                     (MSUB, LANES), 0)
           + jax.lax.broadcasted_iota(jnp.int32, (MSUB, LANES), 1))
    tile = jnp.where(lin < N, tile, jnp.inf)
    sn2_ref[pl.ds(pl.multiple_of(g * MSUB, MSUB), MSUB), :] = tile

    @pl.when(g == MCH - 1)
    def _select():
        bits = jax.lax.bitcast_convert_type(sn2_ref[...], jnp.int32)
        k0 = N // 2 - 1
        k1 = N // 2

        def body(i, carry):
            p0, p1 = carry
            b = 30 - i
            c0 = p0 | (1 << b)
            c1 = p1 | (1 << b)
            n0 = jnp.sum((bits < c0).astype(jnp.int32))
            n1 = jnp.sum((bits < c1).astype(jnp.int32))
            return (jnp.where(n0 <= k0, c0, p0), jnp.where(n1 <= k1, c1, p1))

        t0, t1 = jax.lax.fori_loop(0, 31, body, (0, 0))
        v0 = jax.lax.bitcast_convert_type(t0, jnp.float32)
        v1 = jax.lax.bitcast_convert_type(t1, jnp.float32)
        thr_ref[...] = jnp.full((1, 1), 0.5 * (jnp.sqrt(v0) + jnp.sqrt(v1)),
                                jnp.float32)


def _main_kernel(thr_ref, pos_ref, sc_ref, rot_ref, op_ref, dc_ref,
                 rest_ref, ga_ref, gc_ref, noise_ref, out_ref):
    thr = thr_ref[0, 0]
    gthr = GRAD_THRESHOLD * jnp.maximum(
        gc_ref[...].astype(jnp.float32), 1.0).reshape(BBLK, 1)
    ga = ga_ref[...]
    gn = jnp.sqrt(jnp.sum(ga * ga, axis=1, keepdims=True))
    large = gn >= gthr

    sc = sc_ref[...]
    asc = jnp.exp(sc)
    sn = jnp.sqrt(jnp.sum(asc * asc, axis=1, keepdims=True))
    clone = large & (sn <= thr)
    split = large & (sn > thr)
    act_op = jax.nn.sigmoid(op_ref[...])
    keep = jnp.logical_not((act_op < MIN_OPACITY) | split)

    pos = pos_ref[...]
    P = jnp.concatenate([pos, sc, rot_ref[...], op_ref[...],
                         dc_ref[...], rest_ref[...]], axis=1)
    out_ref[0] = jnp.where(keep, P, 0.0)
    out_ref[1] = jnp.where(clone, P, 0.0)
    for i in range(2):
        front = jnp.concatenate([pos + noise_ref[i] * asc, sc - LOG2],
                                axis=1)
        Pi = jnp.concatenate([front, P[:, 6:23]], axis=1)
        out_ref[2 + i] = jnp.where(split, Pi, 0.0)


def kernel(positions, scales, rotations, opacities, sh_dc, sh_rest,
           grad_accum, grad_count, split_noise):
    f32 = jnp.float32
    thr = pl.pallas_call(
        _median_kernel,
        grid=(MCH,),
        in_specs=[pl.BlockSpec((MB, 3), lambda g: (g, 0))],
        out_specs=pl.BlockSpec((1, 1), lambda g: (0, 0)),
        out_shape=jax.ShapeDtypeStruct((1, 1), f32),
        scratch_shapes=[pltpu.VMEM((MCH * MSUB, LANES), f32)],
    )(scales)

    B = BBLK

    def row_spec(w):
        return pl.BlockSpec((B, w), lambda i: (i, 0))

    out4 = pl.pallas_call(
        _main_kernel,
        grid=(NBLK,),
        in_specs=[
            pl.BlockSpec((1, 1), lambda i: (0, 0)),
            row_spec(3),
            row_spec(3),
            row_spec(4),
            row_spec(1),
            row_spec(3),
            row_spec(9),
            row_spec(2),
            pl.BlockSpec((B,), lambda i: (i,)),
            pl.BlockSpec((2, B, 3), lambda i: (0, i, 0)),
        ],
        out_specs=pl.BlockSpec((4, B, 23), lambda i: (0, i, 0)),
        out_shape=jax.ShapeDtypeStruct((4, N, 23), f32),
    )(thr, positions, scales, rotations, opacities, sh_dc, sh_rest,
      grad_accum, grad_count, split_noise)
    return out4.reshape(4 * N, 23)
